# baseline (speedup 1.0000x reference)
import jax
import jax.numpy as jnp
from jax import lax
from jax.experimental import pallas as pl
from jax.experimental.pallas import tpu as pltpu
from jax.experimental.pallas import tpu_sc as plsc

_N = 10000
_E = 320000
_D = 128
_H = 128
_OUT = 2
_EPS = 1e-5

_NC = 2
_NS = 16
_NW = _NC * _NS
_EW = _E // _NW
_BB = 80
_GB = 21
_NG = 6
_NB = _NG * _GB
_EPAD = _NW * _NB * _BB - _E
_SCRAP = 16
_RPTA = 640
_ZCH = 64
_ZTAIL = _N % _ZCH
_CCH = 80
_LANE = 16


_DEGW = 16


def _make_agg():
  mesh = plsc.VectorSubcoreMesh(core_axis_name="c", subcore_axis_name="s")
  scratch = [
      pltpu.VMEM((_GB, _BB), jnp.int32),
      pltpu.VMEM((_GB, _BB), jnp.int32),
      pltpu.VMEM((_BB, _H), jnp.float32),
      pltpu.VMEM((_BB, _H), jnp.float32),
      pltpu.VMEM((_BB, _H), jnp.float32),
      pltpu.VMEM((_ZCH, _H), jnp.float32),
      pltpu.VMEM_SHARED((_N + _SCRAP, _H), jnp.float32),
      pltpu.SemaphoreType.DMA,
      pltpu.SemaphoreType.DMA,
      pltpu.SemaphoreType.DMA,
      pltpu.SemaphoreType.DMA,
      pltpu.SemaphoreType.DMA,
      pltpu.SemaphoreType.DMA,
  ]

  def body(h_hbm, src_hbm, dst_hbm, out_s, srcv, dstv, rowsa, rowsb, rowsc,
           zbuf, acc, semga, semgb, semgc, semsa, semsb, semsc):
    c = lax.axis_index("c")
    s = lax.axis_index("s")
    z16 = jnp.zeros((_LANE,), jnp.float32)

    def zrow(k, carry):
      i = k // (_H // _LANE)
      j = k - i * (_H // _LANE)
      zbuf[i, pl.ds(j * _LANE, _LANE)] = z16
      return carry

    lax.fori_loop(0, _ZCH * (_H // _LANE), zrow, 0)

    rbase = s * _RPTA
    for t in range(_RPTA // _ZCH):
      r = rbase + t * _ZCH

      @pl.when(r + _ZCH <= _N)
      def _():
        pltpu.sync_copy(zbuf, acc.at[pl.ds(r, _ZCH)])

      @pl.when(jnp.logical_and(r < _N, r + _ZCH > _N))
      def _():
        pltpu.sync_copy(zbuf.at[pl.ds(0, _ZTAIL)], acc.at[pl.ds(r, _ZTAIL)])

    plsc.subcore_barrier()

    wid = c * _NS + s
    bufs = ((rowsa, semga, semsa), (rowsb, semgb, semsb),
            (rowsc, semgc, semsc))

    def group(g, carry):
      pltpu.sync_copy(src_hbm.at[wid, g], srcv)
      pltpu.sync_copy(dst_hbm.at[wid, g], dstv)
      for b, (rows, semg, _) in enumerate(bufs):
        pltpu.async_copy(h_hbm.at[srcv.at[b]], rows, semg)

      def ring(k, carry2):
        j = 3 * k
        for b, (rows, semg, sems) in enumerate(bufs):
          pltpu.make_async_copy(h_hbm.at[srcv.at[j + b]], rows, semg).wait()
          pltpu.async_copy(rows, acc.at[dstv.at[j + b]], sems, add=True)
        for b, (rows, semg, sems) in enumerate(bufs):
          pltpu.make_async_copy(rows, acc.at[dstv.at[j + b]], sems).wait()
          pltpu.async_copy(h_hbm.at[srcv.at[j + 3 + b]], rows, semg)
        return carry2

      lax.fori_loop(0, _GB // 3 - 1, ring, 0)
      jt = _GB - 3
      for b, (rows, semg, sems) in enumerate(bufs):
        pltpu.make_async_copy(h_hbm.at[srcv.at[jt + b]], rows, semg).wait()
        pltpu.async_copy(rows, acc.at[dstv.at[jt + b]], sems, add=True)
      for b, (rows, semg, sems) in enumerate(bufs):
        pltpu.make_async_copy(rows, acc.at[dstv.at[jt + b]], sems).wait()
      return carry

    lax.fori_loop(0, _NG, group, 0)

    plsc.subcore_barrier()

    def copy_out(j, carry):
      r = rbase + j * _CCH

      @pl.when(r < _N)
      def _():
        pltpu.sync_copy(acc.at[pl.ds(r, _CCH)], out_s.at[c, pl.ds(r, _CCH)])

      return carry

    lax.fori_loop(0, _RPTA // _CCH, copy_out, 0)

  return pl.kernel(
      body,
      out_type=jax.ShapeDtypeStruct((_NC, _N, _H), jnp.float32),
      mesh=mesh,
      scratch_types=scratch,
  )


def _make_deg():
  mesh = plsc.VectorSubcoreMesh(core_axis_name="c", subcore_axis_name="s")
  scratch = [
      pltpu.VMEM((_GB, _BB), jnp.int32),
      pltpu.VMEM((_BB, _H), jnp.float32),
      pltpu.VMEM((_ZCH, _H), jnp.float32),
      pltpu.VMEM_SHARED((_N + _SCRAP, _H), jnp.float32),
  ]

  def body(dst_hbm, out_d, dstv, onesv, zbuf, accd):
    c = lax.axis_index("c")
    s = lax.axis_index("s")
    wid = c * _NS + s
    z16 = jnp.zeros((_LANE,), jnp.float32)
    one16 = jnp.ones((_LANE,), jnp.float32)

    def fillrow(k, carry):
      i = k // (_H // _LANE)
      j = k - i * (_H // _LANE)
      onesv[i, pl.ds(j * _LANE, _LANE)] = one16
      return carry

    lax.fori_loop(0, _BB * (_H // _LANE), fillrow, 0)

    def zrow(k, carry):
      i = k // (_H // _LANE)
      j = k - i * (_H // _LANE)
      zbuf[i, pl.ds(j * _LANE, _LANE)] = z16
      return carry

    lax.fori_loop(0, _ZCH * (_H // _LANE), zrow, 0)

    rbase = s * _RPTA
    for t in range(_RPTA // _ZCH):
      r = rbase + t * _ZCH

      @pl.when(r + _ZCH <= _N)
      def _():
        pltpu.sync_copy(zbuf, accd.at[pl.ds(r, _ZCH)])

      @pl.when(jnp.logical_and(r < _N, r + _ZCH > _N))
      def _():
        pltpu.sync_copy(zbuf.at[pl.ds(0, _ZTAIL)], accd.at[pl.ds(r, _ZTAIL)])

    plsc.subcore_barrier()

    def group(g, carry):
      pltpu.sync_copy(dst_hbm.at[wid, g], dstv)

      def step(j, carry2):
        pltpu.sync_copy(onesv, accd.at[dstv.at[j]], add=True)
        return carry2

      lax.fori_loop(0, _GB, step, 0)
      return carry

    lax.fori_loop(0, _NG, group, 0)

    plsc.subcore_barrier()

    def copy_out(j, carry):
      r = rbase + j * _CCH

      @pl.when(r < _N)
      def _():
        pltpu.sync_copy(accd.at[pl.ds(r, _CCH)], out_d.at[c, pl.ds(r, _CCH)])

      return carry

    lax.fori_loop(0, _RPTA // _CCH, copy_out, 0)

  return pl.kernel(
      body,
      out_type=jax.ShapeDtypeStruct((_NC, _N, _H), jnp.float32),
      mesh=mesh,
      scratch_types=scratch,
  )


_sc_cache = {}


def _agg(h, src, dst):
  if "agg" not in _sc_cache:
    _sc_cache["agg"] = _make_agg()
  return _sc_cache["agg"](h, src, dst)


def _deg(dst):
  if "deg" not in _sc_cache:
    _sc_cache["deg"] = _make_deg()
  return _sc_cache["deg"](dst)

_NBLK = 1000


def _deg_inv(d):
  dsum = jnp.sum(d[...], axis=-1, keepdims=True) * (1.0 / _DEGW)
  return 1.0 / jnp.maximum(dsum, 1.0)


def _layer_body(s0, s1, d, h, wl, wr, cc, out):
  agg = (s0[0] + s1[0]) * _deg_inv(d)
  z = (jnp.dot(agg, wl[...], preferred_element_type=jnp.float32)
       + jnp.dot(h[...], wr[...], preferred_element_type=jnp.float32)
       + cc[...])
  out[...] = jnp.maximum(z, 0.0)


def _layer2_body(s0, s1, d, h, wl, wr, cc, wc, bc, out_h, out_o):
  agg = (s0[0] + s1[0]) * _deg_inv(d)
  z = (jnp.dot(agg, wl[...], preferred_element_type=jnp.float32)
       + jnp.dot(h[...], wr[...], preferred_element_type=jnp.float32)
       + cc[...])
  out_h[...] = z
  out_o[...] = jnp.dot(z, wc[...], preferred_element_type=jnp.float32) + bc[...]


def _common_specs():
  return [
      pl.BlockSpec((1, _NBLK, _H), lambda i: (0, i, 0)),
      pl.BlockSpec((1, _NBLK, _H), lambda i: (1, i, 0)),
      pl.BlockSpec((_NBLK, _NC * _DEGW), lambda i: (i, 0)),
      pl.BlockSpec((_NBLK, _H), lambda i: (i, 0)),
      pl.BlockSpec((_H, _H), lambda i: (0, 0)),
      pl.BlockSpec((_H, _H), lambda i: (0, 0)),
      pl.BlockSpec((1, _H), lambda i: (0, 0)),
  ]


def _run_layer(s, degt, h, wl, wr, cc):
  return pl.pallas_call(
      _layer_body,
      grid=(_N // _NBLK,),
      in_specs=_common_specs(),
      out_specs=pl.BlockSpec((_NBLK, _H), lambda i: (i, 0)),
      out_shape=jax.ShapeDtypeStruct((_N, _H), jnp.float32),
  )(s, s, degt, h, wl, wr, cc)


def _run_layer2(s, degt, h, wl, wr, cc, wc, bc):
  return pl.pallas_call(
      _layer2_body,
      grid=(_N // _NBLK,),
      in_specs=_common_specs() + [
          pl.BlockSpec((_H, _H), lambda i: (0, 0)),
          pl.BlockSpec((1, _H), lambda i: (0, 0)),
      ],
      out_specs=[
          pl.BlockSpec((_NBLK, _H), lambda i: (i, 0)),
          pl.BlockSpec((_NBLK, _H), lambda i: (i, 0)),
      ],
      out_shape=[
          jax.ShapeDtypeStruct((_N, _H), jnp.float32),
          jax.ShapeDtypeStruct((_N, _H), jnp.float32),
      ],
  )(s, s, degt, h, wl, wr, cc, wc, bc)


def kernel(x, edge_index, Wl0, bl0, Wr0, g0, b0, Wl1, bl1, Wr1, g1, b1,
           Wl2, bl2, Wr2, g2, b2, Wc, bc):
  src = jnp.concatenate(
      [edge_index[0], jnp.zeros((_EPAD,), jnp.int32)]
  ).reshape(_NW, _NG, _GB, _BB)
  dst = jnp.concatenate(
      [edge_index[1], jnp.full((_EPAD,), _N, jnp.int32)]
  ).reshape(_NW, _NG, _GB, _BB)
  bn_scale = 1.0 / jnp.sqrt(1.0 + _EPS)

  def fold(Wl, bl, Wr, g, b):
    sc = g * bn_scale
    return Wl * sc[None, :], Wr * sc[None, :], (bl * sc + b).reshape(1, _H)

  wl0, wr0, c0 = fold(Wl0, bl0, Wr0, g0, b0)
  wl1, wr1, c1 = fold(Wl1, bl1, Wr1, g1, b1)
  wl2, wr2, c2 = fold(Wl2, bl2, Wr2, g2, b2)
  wcp = jnp.pad(Wc, ((0, 0), (0, _H - _OUT)))
  bcp = jnp.pad(bc, (0, _H - _OUT)).reshape(1, _H)

  deg = _deg(dst)
  degt = jnp.concatenate([deg[0, :, :_DEGW], deg[1, :, :_DEGW]], axis=-1)
  s0 = _agg(x, src, dst)
  h1 = _run_layer(s0, degt, x, wl0, wr0, c0)
  s1 = _agg(h1, src, dst)
  h2 = _run_layer(s1, degt, h1, wl1, wr1, c1)
  s2 = _agg(h2, src, dst)
  h3, outp = _run_layer2(s2, degt, h2, wl2, wr2, c2, wcp, bcp)
  return outp[:, :_OUT], h3


# speedup vs baseline: 1.6772x; 1.6772x over previous
import jax
import jax.numpy as jnp
from jax import lax
from jax.experimental import pallas as pl
from jax.experimental.pallas import tpu as pltpu
from jax.experimental.pallas import tpu_sc as plsc

_N = 10000
_E = 320000
_D = 128
_H = 128
_OUT = 2
_EPS = 1e-5

_NC = 2
_NS = 16
_NW = _NC * _NS
_BB = 80
_GB = 25
_NG = 5
_SCRAP = 16
_RPTA = 640
_ZCH = 64
_ZTAIL = _N % _ZCH
_CCH = 80
_DEGW = 16
_LANE = 16


def _make_agg():
  mesh = plsc.VectorSubcoreMesh(core_axis_name="c", subcore_axis_name="s")
  scratch = [
      pltpu.VMEM((_GB, _BB), jnp.int32),
      pltpu.VMEM((_GB, _BB), jnp.int32),
      pltpu.VMEM((_BB, _H), jnp.float32),
      pltpu.VMEM((_BB, _H), jnp.float32),
      pltpu.VMEM((_ZCH, _H), jnp.float32),
      pltpu.VMEM_SHARED((_N + _SCRAP, _H), jnp.float32),
      pltpu.SemaphoreType.DMA,
      pltpu.SemaphoreType.DMA,
  ]

  def body(h_hbm, src_hbm, dst_hbm, out_s, srcv, dstv, rowsa, rowsb, zbuf,
           acc, sema, semb):
    c = lax.axis_index("c")
    s = lax.axis_index("s")
    wid = c * _NS + s
    z16 = jnp.zeros((_LANE,), jnp.float32)

    def zrow(k, carry):
      i = k // (_H // _LANE)
      j = k - i * (_H // _LANE)
      zbuf[i, pl.ds(j * _LANE, _LANE)] = z16
      return carry

    lax.fori_loop(0, _ZCH * (_H // _LANE), zrow, 0)

    rbase = s * _RPTA
    for t in range(_RPTA // _ZCH):
      r = rbase + t * _ZCH

      @pl.when(r + _ZCH <= _N)
      def _():
        pltpu.sync_copy(zbuf, acc.at[pl.ds(r, _ZCH)])

      @pl.when(jnp.logical_and(r < _N, r + _ZCH > _N))
      def _():
        pltpu.sync_copy(zbuf.at[pl.ds(0, _ZTAIL)], acc.at[pl.ds(r, _ZTAIL)])

    plsc.subcore_barrier()

    def group(g, carry):
      pltpu.sync_copy(src_hbm.at[wid, g], srcv)
      pltpu.sync_copy(dst_hbm.at[wid, g], dstv)
      pltpu.async_copy(h_hbm.at[srcv.at[0]], rowsa, sema)

      def pair(p, carry2):
        j0 = 2 * p
        pltpu.async_copy(h_hbm.at[srcv.at[j0 + 1]], rowsb, semb)
        pltpu.make_async_copy(h_hbm.at[srcv.at[j0]], rowsa, sema).wait()
        pltpu.sync_copy(rowsa, acc.at[dstv.at[j0]], add=True)
        pltpu.async_copy(h_hbm.at[srcv.at[j0 + 2]], rowsa, sema)
        pltpu.make_async_copy(h_hbm.at[srcv.at[j0 + 1]], rowsb, semb).wait()
        pltpu.sync_copy(rowsb, acc.at[dstv.at[j0 + 1]], add=True)
        return carry2

      lax.fori_loop(0, _GB // 2, pair, 0)
      pltpu.make_async_copy(h_hbm.at[srcv.at[_GB - 1]], rowsa, sema).wait()
      pltpu.sync_copy(rowsa, acc.at[dstv.at[_GB - 1]], add=True)
      return carry

    lax.fori_loop(0, _NG, group, 0)

    plsc.subcore_barrier()

    def copy_out(j, carry):
      r = rbase + j * _CCH

      @pl.when(r < _N)
      def _():
        pltpu.sync_copy(acc.at[pl.ds(r, _CCH)], out_s.at[c, pl.ds(r, _CCH)])

      return carry

    lax.fori_loop(0, _RPTA // _CCH, copy_out, 0)

  return pl.kernel(
      body,
      out_type=jax.ShapeDtypeStruct((_NC, _N, _H), jnp.float32),
      mesh=mesh,
      scratch_types=scratch,
  )


def _make_deg():
  mesh = plsc.VectorSubcoreMesh(core_axis_name="c", subcore_axis_name="s")
  scratch = [
      pltpu.VMEM((_GB, _BB), jnp.int32),
      pltpu.VMEM((_BB, _H), jnp.float32),
      pltpu.VMEM((_ZCH, _H), jnp.float32),
      pltpu.VMEM_SHARED((_N + _SCRAP, _H), jnp.float32),
  ]

  def body(dst_hbm, out_d, dstv, onesv, zbuf, accd):
    c = lax.axis_index("c")
    s = lax.axis_index("s")
    wid = c * _NS + s
    z16 = jnp.zeros((_LANE,), jnp.float32)
    one16 = jnp.ones((_LANE,), jnp.float32)

    def fillrow(k, carry):
      i = k // (_H // _LANE)
      j = k - i * (_H // _LANE)
      onesv[i, pl.ds(j * _LANE, _LANE)] = one16
      return carry

    lax.fori_loop(0, _BB * (_H // _LANE), fillrow, 0)

    def zrow(k, carry):
      i = k // (_H // _LANE)
      j = k - i * (_H // _LANE)
      zbuf[i, pl.ds(j * _LANE, _LANE)] = z16
      return carry

    lax.fori_loop(0, _ZCH * (_H // _LANE), zrow, 0)

    rbase = s * _RPTA
    for t in range(_RPTA // _ZCH):
      r = rbase + t * _ZCH

      @pl.when(r + _ZCH <= _N)
      def _():
        pltpu.sync_copy(zbuf, accd.at[pl.ds(r, _ZCH)])

      @pl.when(jnp.logical_and(r < _N, r + _ZCH > _N))
      def _():
        pltpu.sync_copy(zbuf.at[pl.ds(0, _ZTAIL)], accd.at[pl.ds(r, _ZTAIL)])

    plsc.subcore_barrier()

    def group(g, carry):
      pltpu.sync_copy(dst_hbm.at[wid, g], dstv)

      def step(j, carry2):
        pltpu.sync_copy(onesv, accd.at[dstv.at[j]], add=True)
        return carry2

      lax.fori_loop(0, _GB, step, 0)
      return carry

    lax.fori_loop(0, _NG, group, 0)

    plsc.subcore_barrier()

    def copy_out(j, carry):
      r = rbase + j * _CCH

      @pl.when(r < _N)
      def _():
        pltpu.sync_copy(accd.at[pl.ds(r, _CCH)], out_d.at[c, pl.ds(r, _CCH)])

      return carry

    lax.fori_loop(0, _RPTA // _CCH, copy_out, 0)

  return pl.kernel(
      body,
      out_type=jax.ShapeDtypeStruct((_NC, _N, _H), jnp.float32),
      mesh=mesh,
      scratch_types=scratch,
  )


_sc_cache = {}


def _agg(h, src, dst):
  if "agg" not in _sc_cache:
    _sc_cache["agg"] = _make_agg()
  return _sc_cache["agg"](h, src, dst)


def _deg(dst):
  if "deg" not in _sc_cache:
    _sc_cache["deg"] = _make_deg()
  return _sc_cache["deg"](dst)


_NBLK = 1000


def _deg_inv(d):
  dsum = jnp.sum(d[...], axis=-1, keepdims=True) * (1.0 / _DEGW)
  return 1.0 / jnp.maximum(dsum, 1.0)


def _layer_body(s0, s1, d, h, wl, wr, cc, out):
  agg = (s0[0] + s1[0]) * _deg_inv(d)
  z = (jnp.dot(agg, wl[...], preferred_element_type=jnp.float32)
       + jnp.dot(h[...], wr[...], preferred_element_type=jnp.float32)
       + cc[...])
  out[...] = jnp.maximum(z, 0.0)


def _layer2_body(s0, s1, d, h, wl, wr, cc, wc, bc, out_h, out_o):
  agg = (s0[0] + s1[0]) * _deg_inv(d)
  z = (jnp.dot(agg, wl[...], preferred_element_type=jnp.float32)
       + jnp.dot(h[...], wr[...], preferred_element_type=jnp.float32)
       + cc[...])
  out_h[...] = z
  out_o[...] = jnp.dot(z, wc[...], preferred_element_type=jnp.float32) + bc[...]


def _common_specs():
  return [
      pl.BlockSpec((1, _NBLK, _H), lambda i: (0, i, 0)),
      pl.BlockSpec((1, _NBLK, _H), lambda i: (1, i, 0)),
      pl.BlockSpec((_NBLK, _NC * _DEGW), lambda i: (i, 0)),
      pl.BlockSpec((_NBLK, _H), lambda i: (i, 0)),
      pl.BlockSpec((_H, _H), lambda i: (0, 0)),
      pl.BlockSpec((_H, _H), lambda i: (0, 0)),
      pl.BlockSpec((1, _H), lambda i: (0, 0)),
  ]


def _run_layer(s, degt, h, wl, wr, cc):
  return pl.pallas_call(
      _layer_body,
      grid=(_N // _NBLK,),
      in_specs=_common_specs(),
      out_specs=pl.BlockSpec((_NBLK, _H), lambda i: (i, 0)),
      out_shape=jax.ShapeDtypeStruct((_N, _H), jnp.float32),
  )(s, s, degt, h, wl, wr, cc)


def _run_layer2(s, degt, h, wl, wr, cc, wc, bc):
  return pl.pallas_call(
      _layer2_body,
      grid=(_N // _NBLK,),
      in_specs=_common_specs() + [
          pl.BlockSpec((_H, _H), lambda i: (0, 0)),
          pl.BlockSpec((1, _H), lambda i: (0, 0)),
      ],
      out_specs=[
          pl.BlockSpec((_NBLK, _H), lambda i: (i, 0)),
          pl.BlockSpec((_NBLK, _H), lambda i: (i, 0)),
      ],
      out_shape=[
          jax.ShapeDtypeStruct((_N, _H), jnp.float32),
          jax.ShapeDtypeStruct((_N, _H), jnp.float32),
      ],
  )(s, s, degt, h, wl, wr, cc, wc, bc)


def kernel(x, edge_index, Wl0, bl0, Wr0, g0, b0, Wl1, bl1, Wr1, g1, b1,
           Wl2, bl2, Wr2, g2, b2, Wc, bc):
  src = edge_index[0].reshape(_NW, _NG, _GB, _BB)
  dst = edge_index[1].reshape(_NW, _NG, _GB, _BB)
  bn_scale = 1.0 / jnp.sqrt(1.0 + _EPS)

  def fold(Wl, bl, Wr, g, b):
    sc = g * bn_scale
    return Wl * sc[None, :], Wr * sc[None, :], (bl * sc + b).reshape(1, _H)

  wl0, wr0, c0 = fold(Wl0, bl0, Wr0, g0, b0)
  wl1, wr1, c1 = fold(Wl1, bl1, Wr1, g1, b1)
  wl2, wr2, c2 = fold(Wl2, bl2, Wr2, g2, b2)
  wcp = jnp.pad(Wc, ((0, 0), (0, _H - _OUT)))
  bcp = jnp.pad(bc, (0, _H - _OUT)).reshape(1, _H)

  deg = _deg(dst)
  degt = jnp.concatenate([deg[0, :, :_DEGW], deg[1, :, :_DEGW]], axis=-1)
  s0 = _agg(x, src, dst)
  h1 = _run_layer(s0, degt, x, wl0, wr0, c0)
  s1 = _agg(h1, src, dst)
  h2 = _run_layer(s1, degt, h1, wl1, wr1, c1)
  s2 = _agg(h2, src, dst)
  h3, outp = _run_layer2(s2, degt, h2, wl2, wr2, c2, wcp, bcp)
  return outp[:, :_OUT], h3
